# baseline (speedup 1.0000x reference)
import functools

import jax
import jax.numpy as jnp
from jax import lax
from jax.experimental import pallas as pl
from jax.experimental.pallas import tpu as pltpu
from jax.experimental.pallas import tpu_sc as plsc

_N = 10000
_E = 320000
_D = 128
_NC = 2
_NS = 16
_NW = _NC * _NS
_L = 16

_CHUNK = 128
_ITERS = 80
_EPAD = _NW * _ITERS * _CHUNK
_STRIPE = 624


def _mm_body(x_ref, w_ref, o_ref):
    o_ref[...] = jnp.dot(x_ref[...], w_ref[...],
                         preferred_element_type=jnp.float32)


def _matmul(x, w):
    return pl.pallas_call(
        _mm_body,
        grid=(5,),
        in_specs=[
            pl.BlockSpec((2000, _D), lambda i: (i, 0)),
            pl.BlockSpec((_D, _D), lambda i: (0, 0)),
        ],
        out_specs=pl.BlockSpec((2000, _D), lambda i: (i, 0)),
        out_shape=jax.ShapeDtypeStruct((_N, _D), jnp.float32),
    )(x, w)


_mesh = plsc.VectorSubcoreMesh(core_axis_name="c", subcore_axis_name="s")


@functools.partial(
    pl.kernel,
    out_type=jax.ShapeDtypeStruct((_NC, _N, _D), jnp.float32),
    mesh=_mesh,
    scratch_types=[
        pltpu.VMEM((_ITERS // 2, _CHUNK), jnp.int32),
        pltpu.VMEM((_ITERS // 2, _CHUNK), jnp.int32),
        pltpu.VMEM((_ITERS // 2, _CHUNK), jnp.float32),
        pltpu.VMEM((_CHUNK, _D), jnp.float32),
        pltpu.VMEM((_CHUNK, _D), jnp.float32),
        pltpu.VMEM_SHARED((_N, _D), jnp.float32),
        pltpu.SemaphoreType.DMA,
        pltpu.SemaphoreType.DMA,
        pltpu.SemaphoreType.DMA,
        pltpu.SemaphoreType.DMA,
        pltpu.SemaphoreType.DMA,
    ],
)
def _spmm(src_hbm, dst_hbm, ew_hbm, sup_hbm, out_hbm,
          src_all, dst_all, ew_all, rows0, rows1, acc,
          ld_sem, gat0, gat1, scat0, scat1):
    c = lax.axis_index("c")
    s = lax.axis_index("s")
    wid = s * _NC + c
    rows = (rows0, rows1)
    gat = (gat0, gat1)
    scat = (scat0, scat1)
    half = _ITERS // 2

    def _start_slab(h):
        sl = pl.ds(h * half, half)
        a = pltpu.async_copy(src_hbm.at[wid, sl], src_all, ld_sem)
        b = pltpu.async_copy(dst_hbm.at[wid, sl], dst_all, ld_sem)
        d = pltpu.async_copy(ew_hbm.at[wid, sl], ew_all, ld_sem)
        return a, b, d

    slab_descs = _start_slab(0)

    def _zero_row(i, carry):
        for j in range(_D // _L):
            rows0[i, pl.ds(j * _L, _L)] = jnp.zeros((_L,), jnp.float32)
        return carry
    lax.fori_loop(0, _CHUNK, _zero_row, 0)

    stripe = s * _STRIPE
    for off in (0, 128, 256, 384, 496):
        pltpu.sync_copy(rows0, acc.at[pl.ds(stripe + off, _CHUNK)])

    @pl.when(s == _NS - 1)
    def _zero_tail():
        pltpu.sync_copy(rows0, acc.at[pl.ds(_N - _CHUNK, _CHUNK)])
    plsc.subcore_barrier()

    def _start_gather(b, it):
        return pltpu.async_copy(sup_hbm.at[src_all.at[it]], rows[b], gat[b])

    def _wait_gather(b, it):
        pltpu.make_async_copy(sup_hbm.at[src_all.at[it]], rows[b],
                              gat[b]).wait()

    def _start_scatter(b, it):
        pltpu.async_copy(rows[b], acc.at[dst_all.at[it]], scat[b], add=True)

    def _wait_scatter(b, it):
        pltpu.make_async_copy(rows[b], acc.at[dst_all.at[it]], scat[b]).wait()

    def _scale(b, it):
        rb = rows[b]

        def _scale16(g, carry):
            wvec = ew_all[it, pl.ds(g * _L, _L)]
            for l in range(_L):
                wl = wvec.at[jnp.full((_L,), l, jnp.int32)].get(
                    mode="promise_in_bounds")
                r = g * _L + l
                for j in range(_D // _L):
                    sl = pl.ds(j * _L, _L)
                    rb[r, sl] = rb[r, sl] * wl
            return carry
        lax.fori_loop(0, _CHUNK // _L, _scale16, 0)

    def _body(t, carry):
        j0 = t * 2
        j1 = j0 + 1

        @pl.when(t > 0)
        def _():
            _wait_scatter(1, j0 - 1)
        _start_gather(1, j1)
        _wait_gather(0, j0)
        _scale(0, j0)
        _start_scatter(0, j0)

        @pl.when(t < half // 2 - 1)
        def _():
            _wait_scatter(0, j0)
            _start_gather(0, j1 + 1)
        _wait_gather(1, j1)
        _scale(1, j1)
        _start_scatter(1, j1)
        return carry

    for h in range(2):
        for d in slab_descs:
            d.wait()
        _start_gather(0, 0)
        lax.fori_loop(0, half // 2, _body, 0)
        _wait_scatter(0, half - 2)
        _wait_scatter(1, half - 1)
        if h == 0:
            slab_descs = _start_slab(1)

    plsc.subcore_barrier()
    for off in (0, 128, 256, 384, 496):
        pltpu.sync_copy(acc.at[pl.ds(stripe + off, _CHUNK)],
                        out_hbm.at[c, pl.ds(stripe + off, _CHUNK)])

    @pl.when(s == _NS - 1)
    def _write_tail():
        pltpu.sync_copy(acc.at[pl.ds(_N - _CHUNK, _CHUNK)],
                        out_hbm.at[c, pl.ds(_N - _CHUNK, _CHUNK)])


def _comb_body(p_ref, b_ref, o_ref):
    o_ref[...] = p_ref[0] + p_ref[1] + b_ref[...]


def _combine(partials, bias2d):
    return pl.pallas_call(
        _comb_body,
        grid=(5,),
        in_specs=[
            pl.BlockSpec((_NC, 2000, _D), lambda i: (0, i, 0)),
            pl.BlockSpec((1, _D), lambda i: (0, 0)),
        ],
        out_specs=pl.BlockSpec((2000, _D), lambda i: (i, 0)),
        out_shape=jax.ShapeDtypeStruct((_N, _D), jnp.float32),
    )(partials, bias2d)


def kernel(x, edge_index, edge_weight, weight, bias):
    support = _matmul(x, weight)
    pad = _EPAD - _E
    ei = jnp.pad(edge_index, ((0, 0), (0, pad)))
    src3 = ei[0].reshape(_NW, _ITERS, _CHUNK)
    dst3 = ei[1].reshape(_NW, _ITERS, _CHUNK)
    ew3 = jnp.pad(edge_weight, (0, pad)).reshape(_NW, _ITERS, _CHUNK)
    partials = _spmm(src3, dst3, ew3, support)
    return _combine(partials, bias.reshape(1, _D))


# speedup vs baseline: 1.0007x; 1.0007x over previous
import functools

import jax
import jax.numpy as jnp
from jax import lax
from jax.experimental import pallas as pl
from jax.experimental.pallas import tpu as pltpu
from jax.experimental.pallas import tpu_sc as plsc

_N = 10000
_E = 320000
_D = 128
_NC = 2
_NS = 16
_NW = _NC * _NS
_L = 16

_CHUNK = 128
_ITERS = 80
_EPAD = _NW * _ITERS * _CHUNK
_STRIPE = 624


def _mm_body(x_ref, w_ref, o_ref):
    o_ref[...] = jnp.dot(x_ref[...], w_ref[...],
                         preferred_element_type=jnp.float32)


def _matmul(x, w):
    return pl.pallas_call(
        _mm_body,
        grid=(5,),
        in_specs=[
            pl.BlockSpec((2000, _D), lambda i: (i, 0)),
            pl.BlockSpec((_D, _D), lambda i: (0, 0)),
        ],
        out_specs=pl.BlockSpec((2000, _D), lambda i: (i, 0)),
        out_shape=jax.ShapeDtypeStruct((_N, _D), jnp.float32),
    )(x, w)


_mesh = plsc.VectorSubcoreMesh(core_axis_name="c", subcore_axis_name="s")


@functools.partial(
    pl.kernel,
    out_type=jax.ShapeDtypeStruct((_NC, _N, _D), jnp.float32),
    mesh=_mesh,
    scratch_types=[
        pltpu.VMEM((_ITERS // 2, _CHUNK), jnp.int32),
        pltpu.VMEM((_ITERS // 2, _CHUNK), jnp.int32),
        pltpu.VMEM((_ITERS // 2, _CHUNK), jnp.float32),
        pltpu.VMEM((_CHUNK, _D), jnp.float32),
        pltpu.VMEM((_CHUNK, _D), jnp.float32),
        pltpu.VMEM_SHARED((_N, _D), jnp.float32),
        pltpu.SemaphoreType.DMA,
        pltpu.SemaphoreType.DMA,
        pltpu.SemaphoreType.DMA,
        pltpu.SemaphoreType.DMA,
        pltpu.SemaphoreType.DMA,
    ],
)
def _spmm(src_hbm, dst_hbm, ew_hbm, sup_hbm, out_hbm,
          src_all, dst_all, ew_all, rows0, rows1, acc,
          ld_sem, gat0, gat1, scat0, scat1):
    c = lax.axis_index("c")
    s = lax.axis_index("s")
    wid = s * _NC + c
    rows = (rows0, rows1)
    gat = (gat0, gat1)
    scat = (scat0, scat1)
    half = _ITERS // 2

    def _start_slab(h):
        sl = pl.ds(h * half, half)
        a = pltpu.async_copy(src_hbm.at[wid, sl], src_all, ld_sem)
        b = pltpu.async_copy(dst_hbm.at[wid, sl], dst_all, ld_sem)
        d = pltpu.async_copy(ew_hbm.at[wid, sl], ew_all, ld_sem)
        return a, b, d

    slab_descs = _start_slab(0)

    def _zero_row(i, carry):
        for j in range(_D // _L):
            rows0[i, pl.ds(j * _L, _L)] = jnp.zeros((_L,), jnp.float32)
        return carry
    lax.fori_loop(0, _CHUNK, _zero_row, 0)

    stripe = s * _STRIPE
    for off in (0, 128, 256, 384, 496):
        pltpu.sync_copy(rows0, acc.at[pl.ds(stripe + off, _CHUNK)])

    @pl.when(s == _NS - 1)
    def _zero_tail():
        pltpu.sync_copy(rows0, acc.at[pl.ds(_N - _CHUNK, _CHUNK)])
    plsc.subcore_barrier()

    def _start_gather(b, it):
        return pltpu.async_copy(sup_hbm.at[src_all.at[it]], rows[b], gat[b])

    def _wait_gather(b, it):
        pltpu.make_async_copy(sup_hbm.at[src_all.at[it]], rows[b],
                              gat[b]).wait()

    def _start_scatter(b, it):
        pltpu.async_copy(rows[b], acc.at[dst_all.at[it]], scat[b], add=True)

    def _wait_scatter(b, it):
        pltpu.make_async_copy(rows[b], acc.at[dst_all.at[it]], scat[b]).wait()

    def _scale(b, it):
        rb = rows[b]

        def _scale16(g, carry):
            wvec = ew_all[it, pl.ds(g * _L, _L)]
            for l in range(_L):
                wl = wvec.at[jnp.full((_L,), l, jnp.int32)].get(
                    mode="promise_in_bounds")
                r = g * _L + l
                for j in range(_D // _L):
                    sl = pl.ds(j * _L, _L)
                    rb[r, sl] = rb[r, sl] * wl
            return carry
        lax.fori_loop(0, _CHUNK // _L, _scale16, 0)

    def _body(t, carry):
        j0 = t * 2
        j1 = j0 + 1

        _start_gather(1, j1)
        _wait_gather(0, j0)
        _scale(0, j0)
        pltpu.sync_copy(rows[0], acc.at[dst_all.at[j0]], add=True)

        @pl.when(t < half // 2 - 1)
        def _():
            _start_gather(0, j1 + 1)
        _wait_gather(1, j1)
        _scale(1, j1)
        pltpu.sync_copy(rows[1], acc.at[dst_all.at[j1]], add=True)
        return carry

    for h in range(2):
        for d in slab_descs:
            d.wait()
        _start_gather(0, 0)
        lax.fori_loop(0, half // 2, _body, 0)
        if h == 0:
            slab_descs = _start_slab(1)

    plsc.subcore_barrier()
    for off in (0, 128, 256, 384, 496):
        pltpu.sync_copy(acc.at[pl.ds(stripe + off, _CHUNK)],
                        out_hbm.at[c, pl.ds(stripe + off, _CHUNK)])

    @pl.when(s == _NS - 1)
    def _write_tail():
        pltpu.sync_copy(acc.at[pl.ds(_N - _CHUNK, _CHUNK)],
                        out_hbm.at[c, pl.ds(_N - _CHUNK, _CHUNK)])


def _comb_body(p_ref, b_ref, o_ref):
    o_ref[...] = p_ref[0] + p_ref[1] + b_ref[...]


def _combine(partials, bias2d):
    return pl.pallas_call(
        _comb_body,
        grid=(5,),
        in_specs=[
            pl.BlockSpec((_NC, 2000, _D), lambda i: (0, i, 0)),
            pl.BlockSpec((1, _D), lambda i: (0, 0)),
        ],
        out_specs=pl.BlockSpec((2000, _D), lambda i: (i, 0)),
        out_shape=jax.ShapeDtypeStruct((_N, _D), jnp.float32),
    )(partials, bias2d)


def kernel(x, edge_index, edge_weight, weight, bias):
    support = _matmul(x, weight)
    pad = _EPAD - _E
    ei = jnp.pad(edge_index, ((0, 0), (0, pad)))
    src3 = ei[0].reshape(_NW, _ITERS, _CHUNK)
    dst3 = ei[1].reshape(_NW, _ITERS, _CHUNK)
    ew3 = jnp.pad(edge_weight, (0, pad)).reshape(_NW, _ITERS, _CHUNK)
    partials = _spmm(src3, dst3, ew3, support)
    return _combine(partials, bias.reshape(1, _D))
